# Initial kernel scaffold; baseline (speedup 1.0000x reference)
import jax
import jax.numpy as jnp
from jax.experimental import pallas as pl


def kernel(coords, features):
    raise NotImplementedError("write your pallas kernel here")


# speedup vs baseline: 13.1880x; 13.1880x over previous
import functools

import jax
import jax.numpy as jnp
from jax import lax
from jax.experimental import pallas as pl
from jax.experimental.pallas import tpu as pltpu
from jax.experimental.pallas import tpu_sc as plsc

_K = 16
_RB = 256
_CH = 128


def _topk_body(a_ref, b_ref, bt_ref, idx_ref):
    a = a_ref[0]
    b = b_ref[0]
    bt = bt_ref[0]
    L = b.shape[0]
    inner = lax.dot_general(
        a, b, (((1,), (1,)), ((), ())),
        preferred_element_type=jnp.float32,
    )
    sqa = jnp.sum(a * a, axis=1, keepdims=True)
    sqb = jnp.sum(bt * bt, axis=0, keepdims=True)
    d2 = (sqa + sqb) - 2.0 * inner

    base = pl.program_id(0) * L
    iota = lax.broadcasted_iota(jnp.int32, d2.shape, 1)
    big = jnp.int32(L)
    s = d2
    for kk in range(_K):
        mn = jnp.min(s, axis=1, keepdims=True)
        am = jnp.min(jnp.where(s == mn, iota, big), axis=1, keepdims=True)
        idx_ref[0, :, kk:kk + 1] = am + base
        s = jnp.where(iota == am, jnp.inf, s)


def _topk_indices(coords, interpret=False):
    N, L, D = coords.shape
    coords_t = coords.transpose(0, 2, 1)
    return pl.pallas_call(
        _topk_body,
        grid=(N, L // _RB),
        in_specs=[
            pl.BlockSpec((1, _RB, D), lambda n, r: (n, r, 0)),
            pl.BlockSpec((1, L, D), lambda n, r: (n, 0, 0)),
            pl.BlockSpec((1, D, L), lambda n, r: (n, 0, 0)),
        ],
        out_specs=pl.BlockSpec((1, _RB, _K), lambda n, r: (n, r, 0)),
        out_shape=jax.ShapeDtypeStruct((N, L, _K), jnp.int32),
        interpret=interpret,
    )(coords, coords, coords_t)


def _sc_gather(table, idx2d):
    V, D = table.shape
    n_rows, ch = idx2d.shape
    B = n_rows * ch
    info = plsc.get_sparse_core_info()
    NC, NS = info.num_cores, info.num_subcores
    NW = NC * NS
    rows_per_w = n_rows // NW
    mesh = plsc.VectorSubcoreMesh(core_axis_name="c", subcore_axis_name="s")

    @functools.partial(
        pl.kernel,
        mesh=mesh,
        out_type=jax.ShapeDtypeStruct((B, D), jnp.float32),
        scratch_types=[
            pltpu.VMEM((rows_per_w, ch), jnp.int32),
            pltpu.VMEM((ch, D), jnp.float32),
            pltpu.VMEM((ch, D), jnp.float32),
            pltpu.SemaphoreType.DMA,
            pltpu.SemaphoreType.DMA,
        ],
    )
    def gather_kernel(table_hbm, idx_hbm, out_hbm, idx_v, buf0, buf1, sem0, sem1):
        wid = lax.axis_index("s") * NC + lax.axis_index("c")
        ibase = wid * rows_per_w
        pltpu.sync_copy(idx_hbm.at[pl.ds(ibase, rows_per_w)], idx_v)
        obase = ibase * ch

        def chunk(c, buf, sem):
            pltpu.async_copy(table_hbm.at[idx_v.at[c]], buf, sem).wait()
            pltpu.sync_copy(buf, out_hbm.at[pl.ds(obase + c * ch, ch)])

        def body(c2, _):
            chunk(2 * c2, buf0, sem0)
            chunk(2 * c2 + 1, buf1, sem1)
            return 0

        lax.fori_loop(0, rows_per_w // 2, body, 0)

    return gather_kernel(table, idx2d)


def kernel(coords, features):
    N, L, D = features.shape
    idx = _topk_indices(coords)
    table = features.reshape(N * L, D)
    idx2d = idx.reshape((N * L * _K) // _CH, _CH)
    out = _sc_gather(table, idx2d)
    return out.reshape(N * L, _K, D)
